# baseline (speedup 1.0000x reference)
import functools

import jax
import jax.numpy as jnp
from jax import lax
from jax.experimental import pallas as pl
from jax.experimental.pallas import tpu as pltpu
from jax.experimental.pallas import tpu_sc as plsc

N_TOK = 2048
DIM = 1024
N_EXP = 8
D_FF = 1024
N_PAIR = 2 * N_TOK
BM = 128
NBLK = 40
PAD = NBLK * BM
NW = 32
CHUNK = N_PAIR // NW
TOKW = N_TOK // NW

_MESH = plsc.VectorSubcoreMesh(core_axis_name="c", subcore_axis_name="s",
                               num_cores=2, num_subcores=16)


def _wid():
    return lax.axis_index("s") * 2 + lax.axis_index("c")


def _bi(s):
    return lax.broadcast(jnp.asarray(s, jnp.int32), (16,))


def _router_body(x_ref, wg_ref, e1_ref, e2_ref, w0_ref, w1_ref):
    x = x_ref[...]
    logits = lax.dot_general(x, wg_ref[...], (((1,), (1,)), ((), ())),
                             preferred_element_type=jnp.float32)
    m = jnp.max(logits, axis=1, keepdims=True)
    ex = jnp.exp(logits - m)
    p = ex / jnp.sum(ex, axis=1, keepdims=True)
    iota = lax.broadcasted_iota(jnp.int32, p.shape, 1)
    m1 = jnp.max(p, axis=1, keepdims=True)
    i1 = jnp.min(jnp.where(p == m1, iota, N_EXP), axis=1, keepdims=True)
    p2 = jnp.where(iota == i1, -jnp.inf, p)
    m2 = jnp.max(p2, axis=1, keepdims=True)
    i2 = jnp.min(jnp.where(p2 == m2, iota, N_EXP), axis=1, keepdims=True)
    s = m1 + m2
    e1_ref[...] = i1
    e2_ref[...] = i2
    w0_ref[...] = m1 / s
    w1_ref[...] = m2 / s


def _router(x, Wg):
    return pl.pallas_call(
        _router_body,
        in_specs=[pl.BlockSpec((N_TOK, DIM), lambda: (0, 0)),
                  pl.BlockSpec((N_EXP, DIM), lambda: (0, 0))],
        out_specs=[pl.BlockSpec((N_TOK, 1), lambda: (0, 0))] * 4,
        out_shape=[jax.ShapeDtypeStruct((N_TOK, 1), jnp.int32),
                   jax.ShapeDtypeStruct((N_TOK, 1), jnp.int32),
                   jax.ShapeDtypeStruct((N_TOK, 1), jnp.float32),
                   jax.ShapeDtypeStruct((N_TOK, 1), jnp.float32)],
    )(x, Wg)


def _counts_body(eidx_hbm, cnt_hbm, ev, crow):
    w = _wid()
    pltpu.sync_copy(eidx_hbm.at[pl.ds(w * CHUNK, CHUNK)], ev)
    lane = lax.iota(jnp.int32, 16)
    cnt = jnp.zeros((16,), jnp.int32)
    for i in range(CHUNK // 16):
        v = ev[pl.ds(i * 16, 16)]
        for e in range(N_EXP):
            c = jnp.sum(jnp.where(v == _bi(e), _bi(1), _bi(0)))
            cnt = jnp.where(lane == _bi(e), cnt + _bi(c), cnt)
    crow[...] = cnt
    pltpu.sync_copy(crow, cnt_hbm.at[w])


_SC_PARAMS = pltpu.CompilerParams(needs_layout_passes=False)


@functools.partial(pl.kernel,
                   out_type=jax.ShapeDtypeStruct((NW, 16), jnp.int32),
                   mesh=_MESH,
                   compiler_params=_SC_PARAMS,
                   scratch_types=[pltpu.VMEM((CHUNK,), jnp.int32),
                                  pltpu.VMEM((16,), jnp.int32)])
def _sc_counts(eidx_hbm, cnt_hbm, ev, crow):
    _counts_body(eidx_hbm, cnt_hbm, ev, crow)


def _dispatch_body(eidx_hbm, cnt_hbm, x_hbm, xs_hbm, pos_hbm, sb_hbm,
                   cnts_v, ev, base_v, posm, posl, sbv, rows_v, sem):
    w = _wid()
    pltpu.sync_copy(cnt_hbm, cnts_v)
    pltpu.sync_copy(eidx_hbm.at[pl.ds(w * CHUNK, CHUNK)], ev)
    lane = lax.iota(jnp.int32, 16)
    tot = jnp.zeros((16,), jnp.int32)
    pre = jnp.zeros((16,), jnp.int32)
    for t in range(NW):
        row = cnts_v[t]
        tot = tot + row
        pre = pre + row * _bi(jnp.where(t < w, 1, 0))
    padded = ((tot + _bi(BM - 1)) // _bi(BM)) * _bi(BM)
    starts = plsc.cumsum(padded) - padded
    base = starts + pre
    base_v[...] = base

    @pl.when(w == 0)
    def _():
        sbv[...] = starts // _bi(BM)
        pltpu.sync_copy(sbv, sb_hbm)

    for i in range(CHUNK // 16):
        v = ev[pl.ds(i * 16, 16)]
        bpl = plsc.load_gather(base_v, [v])
        rank = jnp.zeros((16,), jnp.int32)
        binc = jnp.zeros((16,), jnp.int32)
        for e in range(N_EXP):
            msk = v == _bi(e)
            mi = jnp.where(msk, _bi(1), _bi(0))
            cm = plsc.cumsum(mi)
            rank = jnp.where(msk, cm - _bi(1), rank)
            binc = jnp.where(lane == _bi(e), binc + _bi(jnp.sum(mi)), binc)
        pos_v = bpl + rank
        posm[i // 2, pl.ds((i % 2) * 16, 16)] = pos_v
        posl[pl.ds(i * 16, 16)] = pos_v
        base_v[...] = base_v[...] + binc

    pltpu.sync_copy(posl, pos_hbm.at[pl.ds(w * CHUNK, CHUNK)])
    tokbase = (w % 16) * CHUNK
    for c in range(4):
        pltpu.sync_copy(x_hbm.at[pl.ds(tokbase + c * 32, 32)], rows_v)
        pltpu.async_copy(rows_v, xs_hbm.at[posm.at[c]], sem).wait()


@functools.partial(pl.kernel,
                   out_type=[jax.ShapeDtypeStruct((PAD, DIM), jnp.float32),
                             jax.ShapeDtypeStruct((N_PAIR,), jnp.int32),
                             jax.ShapeDtypeStruct((16,), jnp.int32)],
                   mesh=_MESH,
                   compiler_params=_SC_PARAMS,
                   scratch_types=[pltpu.VMEM((NW, 16), jnp.int32),
                                  pltpu.VMEM((CHUNK,), jnp.int32),
                                  pltpu.VMEM((16,), jnp.int32),
                                  pltpu.VMEM((4, 32), jnp.int32),
                                  pltpu.VMEM((CHUNK,), jnp.int32),
                                  pltpu.VMEM((16,), jnp.int32),
                                  pltpu.VMEM((32, DIM), jnp.float32),
                                  pltpu.SemaphoreType.DMA])
def _sc_dispatch(eidx_hbm, cnt_hbm, x_hbm, xs_hbm, pos_hbm, sb_hbm,
                 cnts_v, ev, base_v, posm, posl, sbv, rows_v, sem):
    _dispatch_body(eidx_hbm, cnt_hbm, x_hbm, xs_hbm, pos_hbm, sb_hbm,
                   cnts_v, ev, base_v, posm, posl, sbv, rows_v, sem)


def _cast_body(w1_ref, w2_ref, o1_ref, o2_ref):
    o1_ref[...] = w1_ref[...].astype(jnp.bfloat16)
    o2_ref[...] = w2_ref[...].astype(jnp.bfloat16)


def _cast_weights(W1, W2):
    return pl.pallas_call(
        _cast_body,
        grid=(N_EXP,),
        in_specs=[pl.BlockSpec((1, DIM, D_FF), lambda e: (e, 0, 0)),
                  pl.BlockSpec((1, D_FF, DIM), lambda e: (e, 0, 0))],
        out_specs=[pl.BlockSpec((1, DIM, D_FF), lambda e: (e, 0, 0)),
                   pl.BlockSpec((1, D_FF, DIM), lambda e: (e, 0, 0))],
        out_shape=[jax.ShapeDtypeStruct((N_EXP, DIM, D_FF), jnp.bfloat16),
                   jax.ShapeDtypeStruct((N_EXP, D_FF, DIM), jnp.bfloat16)],
    )(W1, W2)


def _ffn_body(sb_ref, xs_ref, w1_ref, w2_ref, ys_ref):
    b = pl.program_id(0)
    eb = jnp.int32(0)
    for e in range(1, N_EXP):
        eb = eb + jnp.where(sb_ref[e] <= b, 1, 0).astype(jnp.int32)

    @pl.when(b < sb_ref[N_EXP])
    def _():
        xb = xs_ref[...].astype(jnp.bfloat16)
        h = jnp.dot(xb, w1_ref[eb], preferred_element_type=jnp.float32)
        h = h * (1.0 / (1.0 + jnp.exp(-h)))
        ys_ref[...] = jnp.dot(h.astype(jnp.bfloat16), w2_ref[eb],
                              preferred_element_type=jnp.float32)


def _ffn(sb, xs, W1b, W2b):
    grid_spec = pltpu.PrefetchScalarGridSpec(
        num_scalar_prefetch=1,
        grid=(NBLK,),
        in_specs=[
            pl.BlockSpec((BM, DIM), lambda b, sb: (b, 0)),
            pl.BlockSpec((N_EXP, DIM, D_FF), lambda b, sb: (0, 0, 0)),
            pl.BlockSpec((N_EXP, D_FF, DIM), lambda b, sb: (0, 0, 0)),
        ],
        out_specs=pl.BlockSpec((BM, DIM), lambda b, sb: (b, 0)),
    )
    return pl.pallas_call(
        _ffn_body,
        grid_spec=grid_spec,
        out_shape=jax.ShapeDtypeStruct((PAD, DIM), jnp.float32),
    )(sb, xs, W1b, W2b)


def _combine_body(ys_hbm, pos_hbm, w0_hbm, w1_hbm, y_hbm,
                  p0v, p1v, w0v, w1v, a_v, b_v, o_v, sem0, sem1):
    w = _wid()
    for c in range(TOKW // 32):
        tb = w * TOKW + c * 32
        pltpu.sync_copy(pos_hbm.at[pl.ds(tb, 32)], p0v)
        pltpu.sync_copy(pos_hbm.at[pl.ds(N_TOK + tb, 32)], p1v)
        pltpu.sync_copy(w0_hbm.at[pl.ds(tb, 32)], w0v)
        pltpu.sync_copy(w1_hbm.at[pl.ds(tb, 32)], w1v)
        cpa = pltpu.async_copy(ys_hbm.at[p0v], a_v, sem0)
        cpb = pltpu.async_copy(ys_hbm.at[p1v], b_v, sem1)
        cpa.wait()
        cpb.wait()

        def _row(t, carry):
            ts = _bi(t)
            w0s = plsc.load_gather(w0v, [ts])
            w1s = plsc.load_gather(w1v, [ts])
            for j in range(DIM // 16):
                sl = pl.ds(j * 16, 16)
                o_v[t, sl] = a_v[t, sl] * w0s + b_v[t, sl] * w1s
            return carry

        lax.fori_loop(0, 32, _row, 0)
        pltpu.sync_copy(o_v, y_hbm.at[pl.ds(tb, 32)])


@functools.partial(pl.kernel,
                   out_type=jax.ShapeDtypeStruct((N_TOK, DIM), jnp.float32),
                   mesh=_MESH,
                   compiler_params=_SC_PARAMS,
                   scratch_types=[pltpu.VMEM((32,), jnp.int32),
                                  pltpu.VMEM((32,), jnp.int32),
                                  pltpu.VMEM((32,), jnp.float32),
                                  pltpu.VMEM((32,), jnp.float32),
                                  pltpu.VMEM((32, DIM), jnp.float32),
                                  pltpu.VMEM((32, DIM), jnp.float32),
                                  pltpu.VMEM((32, DIM), jnp.float32),
                                  pltpu.SemaphoreType.DMA,
                                  pltpu.SemaphoreType.DMA])
def _sc_combine(ys_hbm, pos_hbm, w0_hbm, w1_hbm, y_hbm,
                p0v, p1v, w0v, w1v, a_v, b_v, o_v, sem0, sem1):
    _combine_body(ys_hbm, pos_hbm, w0_hbm, w1_hbm, y_hbm,
                  p0v, p1v, w0v, w1v, a_v, b_v, o_v, sem0, sem1)


def kernel(x, Wg, W1, W2):
    e1, e2, w0, w1 = _router(x, Wg)
    eidx = jnp.concatenate([e1.reshape(-1), e2.reshape(-1)])
    cnts = _sc_counts(eidx)
    W1b, W2b = _cast_weights(W1, W2)
    xs, pos, sb = _sc_dispatch(eidx, cnts, x)
    ys = _ffn(sb, xs, W1b, W2b)
    return _sc_combine(ys, pos, w0.reshape(-1), w1.reshape(-1))


# speedup vs baseline: 1.1862x; 1.1862x over previous
import functools

import jax
import jax.numpy as jnp
from jax import lax
from jax.experimental import pallas as pl
from jax.experimental.pallas import tpu as pltpu
from jax.experimental.pallas import tpu_sc as plsc

N_TOK = 2048
DIM = 1024
N_EXP = 8
D_FF = 1024
N_PAIR = 2 * N_TOK
BM = 128
NBLK = 40
PAD = NBLK * BM
NW = 32
CHUNK = N_PAIR // NW
TOKW = N_TOK // NW

_MESH = plsc.VectorSubcoreMesh(core_axis_name="c", subcore_axis_name="s",
                               num_cores=2, num_subcores=16)


def _wid():
    return lax.axis_index("s") * 2 + lax.axis_index("c")


def _bi(s):
    return lax.broadcast(jnp.asarray(s, jnp.int32), (16,))


def _router_body(x_ref, wg_ref, e1_ref, e2_ref, w0_ref, w1_ref):
    x = x_ref[...]
    logits = lax.dot_general(x, wg_ref[...], (((1,), (1,)), ((), ())),
                             preferred_element_type=jnp.float32)
    m = jnp.max(logits, axis=1, keepdims=True)
    ex = jnp.exp(logits - m)
    p = ex / jnp.sum(ex, axis=1, keepdims=True)
    iota = lax.broadcasted_iota(jnp.int32, p.shape, 1)
    m1 = jnp.max(p, axis=1, keepdims=True)
    i1 = jnp.min(jnp.where(p == m1, iota, N_EXP), axis=1, keepdims=True)
    p2 = jnp.where(iota == i1, -jnp.inf, p)
    m2 = jnp.max(p2, axis=1, keepdims=True)
    i2 = jnp.min(jnp.where(p2 == m2, iota, N_EXP), axis=1, keepdims=True)
    s = m1 + m2
    e1_ref[...] = i1
    e2_ref[...] = i2
    w0_ref[...] = m1 / s
    w1_ref[...] = m2 / s


def _router(x, Wg):
    return pl.pallas_call(
        _router_body,
        in_specs=[pl.BlockSpec((N_TOK, DIM), lambda: (0, 0)),
                  pl.BlockSpec((N_EXP, DIM), lambda: (0, 0))],
        out_specs=[pl.BlockSpec((N_TOK, 1), lambda: (0, 0))] * 4,
        out_shape=[jax.ShapeDtypeStruct((N_TOK, 1), jnp.int32),
                   jax.ShapeDtypeStruct((N_TOK, 1), jnp.int32),
                   jax.ShapeDtypeStruct((N_TOK, 1), jnp.float32),
                   jax.ShapeDtypeStruct((N_TOK, 1), jnp.float32)],
    )(x, Wg)


def _counts_body(eidx_hbm, cnt_hbm, ev, crow):
    w = _wid()
    pltpu.sync_copy(eidx_hbm.at[pl.ds(w * CHUNK, CHUNK)], ev)
    lane = lax.iota(jnp.int32, 16)
    cnt = jnp.zeros((16,), jnp.int32)
    for i in range(CHUNK // 16):
        v = ev[pl.ds(i * 16, 16)]
        for e in range(N_EXP):
            c = jnp.sum(jnp.where(v == _bi(e), _bi(1), _bi(0)))
            cnt = jnp.where(lane == _bi(e), cnt + _bi(c), cnt)
    crow[...] = cnt
    pltpu.sync_copy(crow, cnt_hbm.at[w])


_SC_PARAMS = pltpu.CompilerParams(needs_layout_passes=False)


@functools.partial(pl.kernel,
                   out_type=jax.ShapeDtypeStruct((NW, 16), jnp.int32),
                   mesh=_MESH,
                   compiler_params=_SC_PARAMS,
                   scratch_types=[pltpu.VMEM((CHUNK,), jnp.int32),
                                  pltpu.VMEM((16,), jnp.int32)])
def _sc_counts(eidx_hbm, cnt_hbm, ev, crow):
    _counts_body(eidx_hbm, cnt_hbm, ev, crow)


def _dispatch_body(eidx_hbm, cnt_hbm, x_hbm, xs_hbm, pos_hbm, sb_hbm,
                   cnts_v, ev, base_v, posm, posl, sbv, rows_v, sem):
    w = _wid()
    pltpu.sync_copy(cnt_hbm, cnts_v)
    pltpu.sync_copy(eidx_hbm.at[pl.ds(w * CHUNK, CHUNK)], ev)
    lane = lax.iota(jnp.int32, 16)
    tot = jnp.zeros((16,), jnp.int32)
    pre = jnp.zeros((16,), jnp.int32)
    for t in range(NW):
        row = cnts_v[t]
        tot = tot + row
        pre = pre + row * _bi(jnp.where(t < w, 1, 0))
    padded = ((tot + _bi(BM - 1)) // _bi(BM)) * _bi(BM)
    starts = plsc.cumsum(padded) - padded
    base = starts + pre
    base_v[...] = base

    @pl.when(w == 0)
    def _():
        sbv[...] = starts // _bi(BM)
        pltpu.sync_copy(sbv, sb_hbm)

    for i in range(CHUNK // 16):
        v = ev[pl.ds(i * 16, 16)]
        bpl = plsc.load_gather(base_v, [v])
        rank = jnp.zeros((16,), jnp.int32)
        binc = jnp.zeros((16,), jnp.int32)
        for e in range(N_EXP):
            msk = v == _bi(e)
            mi = jnp.where(msk, _bi(1), _bi(0))
            cm = plsc.cumsum(mi)
            rank = jnp.where(msk, cm - _bi(1), rank)
            binc = jnp.where(lane == _bi(e), binc + _bi(jnp.sum(mi)), binc)
        pos_v = bpl + rank
        posm[i // 2, pl.ds((i % 2) * 16, 16)] = pos_v
        posl[pl.ds(i * 16, 16)] = pos_v
        base_v[...] = base_v[...] + binc

    pltpu.sync_copy(posl, pos_hbm.at[pl.ds(w * CHUNK, CHUNK)])
    tokbase = (w % 16) * CHUNK
    for c in range(4):
        pltpu.sync_copy(x_hbm.at[pl.ds(tokbase + c * 32, 32)], rows_v)
        pltpu.async_copy(rows_v, xs_hbm.at[posm.at[c]], sem).wait()


@functools.partial(pl.kernel,
                   out_type=[jax.ShapeDtypeStruct((PAD, DIM), jnp.float32),
                             jax.ShapeDtypeStruct((N_PAIR,), jnp.int32),
                             jax.ShapeDtypeStruct((16,), jnp.int32)],
                   mesh=_MESH,
                   compiler_params=_SC_PARAMS,
                   scratch_types=[pltpu.VMEM((NW, 16), jnp.int32),
                                  pltpu.VMEM((CHUNK,), jnp.int32),
                                  pltpu.VMEM((16,), jnp.int32),
                                  pltpu.VMEM((4, 32), jnp.int32),
                                  pltpu.VMEM((CHUNK,), jnp.int32),
                                  pltpu.VMEM((16,), jnp.int32),
                                  pltpu.VMEM((32, DIM), jnp.float32),
                                  pltpu.SemaphoreType.DMA])
def _sc_dispatch(eidx_hbm, cnt_hbm, x_hbm, xs_hbm, pos_hbm, sb_hbm,
                 cnts_v, ev, base_v, posm, posl, sbv, rows_v, sem):
    _dispatch_body(eidx_hbm, cnt_hbm, x_hbm, xs_hbm, pos_hbm, sb_hbm,
                   cnts_v, ev, base_v, posm, posl, sbv, rows_v, sem)


def _silu(h):
    return h * (1.0 / (1.0 + jnp.exp(-h)))


def _ffn_body(sb_ref, xs_ref, w1_ref, w2_ref, ys_ref, hbuf):
    e = pl.program_id(0)
    lo = sb_ref[e]
    hi = sb_ref[e + 1]
    n = hi - lo

    @pl.when(n > 0)
    def _():
        x0 = xs_ref[pl.ds(lo * BM, BM), :]
        h0 = _silu(jnp.dot(x0, w1_ref[0], preferred_element_type=jnp.float32))
        hbuf[0] = h0

        def _step(k, carry):
            b = lo + k
            xb = xs_ref[pl.ds(b * BM, BM), :]
            hb = _silu(jnp.dot(xb, w1_ref[0],
                               preferred_element_type=jnp.float32))
            yb = jnp.dot(hbuf[(k - 1) % 2], w2_ref[0],
                         preferred_element_type=jnp.float32)
            ys_ref[pl.ds((b - 1) * BM, BM), :] = yb
            hbuf[k % 2] = hb
            return carry

        lax.fori_loop(1, n, _step, 0)
        yl = jnp.dot(hbuf[(n - 1) % 2], w2_ref[0],
                     preferred_element_type=jnp.float32)
        ys_ref[pl.ds((hi - 1) * BM, BM), :] = yl


def _ffn(sb, xs, W1, W2):
    grid_spec = pltpu.PrefetchScalarGridSpec(
        num_scalar_prefetch=1,
        grid=(N_EXP,),
        in_specs=[
            pl.BlockSpec((PAD, DIM), lambda e, sb: (0, 0)),
            pl.BlockSpec((1, DIM, D_FF), lambda e, sb: (e, 0, 0)),
            pl.BlockSpec((1, D_FF, DIM), lambda e, sb: (e, 0, 0)),
        ],
        out_specs=pl.BlockSpec((PAD, DIM), lambda e, sb: (0, 0)),
        scratch_shapes=[pltpu.VMEM((2, BM, D_FF), jnp.float32)],
    )
    return pl.pallas_call(
        _ffn_body,
        grid_spec=grid_spec,
        out_shape=jax.ShapeDtypeStruct((PAD, DIM), jnp.float32),
        compiler_params=pltpu.CompilerParams(
            vmem_limit_bytes=64 * 1024 * 1024),
    )(sb, xs, W1, W2)


def _combine_body(ys_hbm, pos_hbm, w0_hbm, w1_hbm, y_hbm,
                  p0v, p1v, w0v, w1v, a_v, b_v, o_v, sem0, sem1):
    w = _wid()
    for c in range(TOKW // 32):
        tb = w * TOKW + c * 32
        pltpu.sync_copy(pos_hbm.at[pl.ds(tb, 32)], p0v)
        pltpu.sync_copy(pos_hbm.at[pl.ds(N_TOK + tb, 32)], p1v)
        pltpu.sync_copy(w0_hbm.at[pl.ds(tb, 32)], w0v)
        pltpu.sync_copy(w1_hbm.at[pl.ds(tb, 32)], w1v)
        cpa = pltpu.async_copy(ys_hbm.at[p0v], a_v, sem0)
        cpb = pltpu.async_copy(ys_hbm.at[p1v], b_v, sem1)
        cpa.wait()
        cpb.wait()

        def _row(t, carry):
            ts = _bi(t)
            w0s = plsc.load_gather(w0v, [ts])
            w1s = plsc.load_gather(w1v, [ts])
            for j in range(DIM // 16):
                sl = pl.ds(j * 16, 16)
                o_v[t, sl] = a_v[t, sl] * w0s + b_v[t, sl] * w1s
            return carry

        lax.fori_loop(0, 32, _row, 0)
        pltpu.sync_copy(o_v, y_hbm.at[pl.ds(tb, 32)])


@functools.partial(pl.kernel,
                   out_type=jax.ShapeDtypeStruct((N_TOK, DIM), jnp.float32),
                   mesh=_MESH,
                   compiler_params=_SC_PARAMS,
                   scratch_types=[pltpu.VMEM((32,), jnp.int32),
                                  pltpu.VMEM((32,), jnp.int32),
                                  pltpu.VMEM((32,), jnp.float32),
                                  pltpu.VMEM((32,), jnp.float32),
                                  pltpu.VMEM((32, DIM), jnp.float32),
                                  pltpu.VMEM((32, DIM), jnp.float32),
                                  pltpu.VMEM((32, DIM), jnp.float32),
                                  pltpu.SemaphoreType.DMA,
                                  pltpu.SemaphoreType.DMA])
def _sc_combine(ys_hbm, pos_hbm, w0_hbm, w1_hbm, y_hbm,
                p0v, p1v, w0v, w1v, a_v, b_v, o_v, sem0, sem1):
    _combine_body(ys_hbm, pos_hbm, w0_hbm, w1_hbm, y_hbm,
                  p0v, p1v, w0v, w1v, a_v, b_v, o_v, sem0, sem1)


def kernel(x, Wg, W1, W2):
    e1, e2, w0, w1 = _router(x, Wg)
    eidx = jnp.concatenate([e1.reshape(-1), e2.reshape(-1)])
    cnts = _sc_counts(eidx)
    xs, pos, sb = _sc_dispatch(eidx, cnts, x)
    ys = _ffn(sb, xs, W1, W2)
    return _sc_combine(ys, pos, w0.reshape(-1), w1.reshape(-1))


# speedup vs baseline: 1.2341x; 1.0404x over previous
import functools

import jax
import jax.numpy as jnp
from jax import lax
from jax.experimental import pallas as pl
from jax.experimental.pallas import tpu as pltpu
from jax.experimental.pallas import tpu_sc as plsc

N_TOK = 2048
DIM = 1024
N_EXP = 8
D_FF = 1024
N_PAIR = 2 * N_TOK
BM = 128
NBLK = 40
PAD = NBLK * BM
NW = 32
CHUNK = N_PAIR // NW
TOKW = N_TOK // NW

_MESH = plsc.VectorSubcoreMesh(core_axis_name="c", subcore_axis_name="s",
                               num_cores=2, num_subcores=16)


def _wid():
    return lax.axis_index("s") * 2 + lax.axis_index("c")


def _bi(s):
    return lax.broadcast(jnp.asarray(s, jnp.int32), (16,))


def _router_body(x_ref, wg_ref, e1_ref, e2_ref, w0_ref, w1_ref):
    x = x_ref[...]
    logits = lax.dot_general(x, wg_ref[...], (((1,), (1,)), ((), ())),
                             preferred_element_type=jnp.float32)
    m = jnp.max(logits, axis=1, keepdims=True)
    ex = jnp.exp(logits - m)
    p = ex / jnp.sum(ex, axis=1, keepdims=True)
    iota = lax.broadcasted_iota(jnp.int32, p.shape, 1)
    m1 = jnp.max(p, axis=1, keepdims=True)
    i1 = jnp.min(jnp.where(p == m1, iota, N_EXP), axis=1, keepdims=True)
    p2 = jnp.where(iota == i1, -jnp.inf, p)
    m2 = jnp.max(p2, axis=1, keepdims=True)
    i2 = jnp.min(jnp.where(p2 == m2, iota, N_EXP), axis=1, keepdims=True)
    s = m1 + m2
    e1_ref[...] = i1
    e2_ref[...] = i2
    w0_ref[...] = m1 / s
    w1_ref[...] = m2 / s


def _router(x, Wg):
    return pl.pallas_call(
        _router_body,
        in_specs=[pl.BlockSpec((N_TOK, DIM), lambda: (0, 0)),
                  pl.BlockSpec((N_EXP, DIM), lambda: (0, 0))],
        out_specs=[pl.BlockSpec((N_TOK, 1), lambda: (0, 0))] * 4,
        out_shape=[jax.ShapeDtypeStruct((N_TOK, 1), jnp.int32),
                   jax.ShapeDtypeStruct((N_TOK, 1), jnp.int32),
                   jax.ShapeDtypeStruct((N_TOK, 1), jnp.float32),
                   jax.ShapeDtypeStruct((N_TOK, 1), jnp.float32)],
    )(x, Wg)


def _counts_body(eidx_hbm, cnt_hbm, ev, crow):
    w = _wid()
    pltpu.sync_copy(eidx_hbm.at[pl.ds(w * CHUNK, CHUNK)], ev)
    lane = lax.iota(jnp.int32, 16)
    cnt = jnp.zeros((16,), jnp.int32)
    for i in range(CHUNK // 16):
        v = ev[pl.ds(i * 16, 16)]
        for e in range(N_EXP):
            c = jnp.sum(jnp.where(v == _bi(e), _bi(1), _bi(0)))
            cnt = jnp.where(lane == _bi(e), cnt + _bi(c), cnt)
    crow[...] = cnt
    pltpu.sync_copy(crow, cnt_hbm.at[w])


_SC_PARAMS = pltpu.CompilerParams(needs_layout_passes=False)


@functools.partial(pl.kernel,
                   out_type=jax.ShapeDtypeStruct((NW, 16), jnp.int32),
                   mesh=_MESH,
                   compiler_params=_SC_PARAMS,
                   scratch_types=[pltpu.VMEM((CHUNK,), jnp.int32),
                                  pltpu.VMEM((16,), jnp.int32)])
def _sc_counts(eidx_hbm, cnt_hbm, ev, crow):
    _counts_body(eidx_hbm, cnt_hbm, ev, crow)


def _dispatch_body(eidx_hbm, cnt_hbm, x_hbm, xs_hbm, pos_hbm, sb_hbm,
                   cnts_v, ev, base_v, posm, posl, sbv, rows_v, sem):
    w = _wid()
    pltpu.sync_copy(cnt_hbm, cnts_v)
    pltpu.sync_copy(eidx_hbm.at[pl.ds(w * CHUNK, CHUNK)], ev)
    lane = lax.iota(jnp.int32, 16)
    tot = jnp.zeros((16,), jnp.int32)
    pre = jnp.zeros((16,), jnp.int32)
    for t in range(NW):
        row = cnts_v[t]
        tot = tot + row
        pre = pre + row * _bi(jnp.where(t < w, 1, 0))
    padded = ((tot + _bi(BM - 1)) // _bi(BM)) * _bi(BM)
    starts = plsc.cumsum(padded) - padded
    base = starts + pre
    base_v[...] = base

    @pl.when(w == 0)
    def _():
        sbv[...] = starts // _bi(BM)
        pltpu.sync_copy(sbv, sb_hbm)

    for i in range(CHUNK // 16):
        v = ev[pl.ds(i * 16, 16)]
        bpl = plsc.load_gather(base_v, [v])
        rank = jnp.zeros((16,), jnp.int32)
        binc = jnp.zeros((16,), jnp.int32)
        for e in range(N_EXP):
            msk = v == _bi(e)
            mi = jnp.where(msk, _bi(1), _bi(0))
            cm = plsc.cumsum(mi)
            rank = jnp.where(msk, cm - _bi(1), rank)
            binc = jnp.where(lane == _bi(e), binc + _bi(jnp.sum(mi)), binc)
        pos_v = bpl + rank
        posm[i // 2, pl.ds((i % 2) * 16, 16)] = pos_v
        posl[pl.ds(i * 16, 16)] = pos_v
        base_v[...] = base_v[...] + binc

    pltpu.sync_copy(posl, pos_hbm.at[pl.ds(w * CHUNK, CHUNK)])
    tokbase = (w % 16) * CHUNK
    for c in range(4):
        pltpu.sync_copy(x_hbm.at[pl.ds(tokbase + c * 32, 32)], rows_v)
        pltpu.async_copy(rows_v, xs_hbm.at[posm.at[c]], sem).wait()


@functools.partial(pl.kernel,
                   out_type=[jax.ShapeDtypeStruct((PAD, DIM), jnp.float32),
                             jax.ShapeDtypeStruct((N_PAIR,), jnp.int32),
                             jax.ShapeDtypeStruct((16,), jnp.int32)],
                   mesh=_MESH,
                   compiler_params=_SC_PARAMS,
                   scratch_types=[pltpu.VMEM((NW, 16), jnp.int32),
                                  pltpu.VMEM((CHUNK,), jnp.int32),
                                  pltpu.VMEM((16,), jnp.int32),
                                  pltpu.VMEM((4, 32), jnp.int32),
                                  pltpu.VMEM((CHUNK,), jnp.int32),
                                  pltpu.VMEM((16,), jnp.int32),
                                  pltpu.VMEM((32, DIM), jnp.float32),
                                  pltpu.SemaphoreType.DMA])
def _sc_dispatch(eidx_hbm, cnt_hbm, x_hbm, xs_hbm, pos_hbm, sb_hbm,
                 cnts_v, ev, base_v, posm, posl, sbv, rows_v, sem):
    _dispatch_body(eidx_hbm, cnt_hbm, x_hbm, xs_hbm, pos_hbm, sb_hbm,
                   cnts_v, ev, base_v, posm, posl, sbv, rows_v, sem)


def _silu(h):
    return h * (1.0 / (1.0 + jnp.exp(-h)))


def _ffn_body(sb_ref, xs_ref, w1_ref, w2_ref, ys_ref):
    e = pl.program_id(0)
    lo = sb_ref[e]
    hi = sb_ref[e + 1]

    def _step(b, carry):
        xb = xs_ref[pl.ds(b * BM, BM), :]
        h = _silu(jnp.dot(xb, w1_ref[0], preferred_element_type=jnp.float32))
        ys_ref[pl.ds(b * BM, BM), :] = jnp.dot(
            h, w2_ref[0], preferred_element_type=jnp.float32)
        return carry

    lax.fori_loop(lo, hi, _step, 0)


def _ffn(sb, xs, W1, W2):
    grid_spec = pltpu.PrefetchScalarGridSpec(
        num_scalar_prefetch=1,
        grid=(N_EXP,),
        in_specs=[
            pl.BlockSpec((PAD, DIM), lambda e, sb: (0, 0)),
            pl.BlockSpec((1, DIM, D_FF), lambda e, sb: (e, 0, 0)),
            pl.BlockSpec((1, D_FF, DIM), lambda e, sb: (e, 0, 0)),
        ],
        out_specs=pl.BlockSpec((PAD, DIM), lambda e, sb: (0, 0)),
    )
    return pl.pallas_call(
        _ffn_body,
        grid_spec=grid_spec,
        out_shape=jax.ShapeDtypeStruct((PAD, DIM), jnp.float32),
    )(sb, xs, W1, W2)


def _combine_body(ys_hbm, pos_hbm, w0_hbm, w1_hbm, y_hbm,
                  p0v, p1v, w0v, w1v, a_v, b_v, o_v, sem0, sem1):
    w = _wid()
    for c in range(TOKW // 32):
        tb = w * TOKW + c * 32
        pltpu.sync_copy(pos_hbm.at[pl.ds(tb, 32)], p0v)
        pltpu.sync_copy(pos_hbm.at[pl.ds(N_TOK + tb, 32)], p1v)
        pltpu.sync_copy(w0_hbm.at[pl.ds(tb, 32)], w0v)
        pltpu.sync_copy(w1_hbm.at[pl.ds(tb, 32)], w1v)
        cpa = pltpu.async_copy(ys_hbm.at[p0v], a_v, sem0)
        cpb = pltpu.async_copy(ys_hbm.at[p1v], b_v, sem1)
        cpa.wait()
        cpb.wait()

        def _row(t, carry):
            ts = _bi(t)
            w0s = plsc.load_gather(w0v, [ts])
            w1s = plsc.load_gather(w1v, [ts])
            for j in range(DIM // 16):
                sl = pl.ds(j * 16, 16)
                o_v[t, sl] = a_v[t, sl] * w0s + b_v[t, sl] * w1s
            return carry

        lax.fori_loop(0, 32, _row, 0)
        pltpu.sync_copy(o_v, y_hbm.at[pl.ds(tb, 32)])


@functools.partial(pl.kernel,
                   out_type=jax.ShapeDtypeStruct((N_TOK, DIM), jnp.float32),
                   mesh=_MESH,
                   compiler_params=_SC_PARAMS,
                   scratch_types=[pltpu.VMEM((32,), jnp.int32),
                                  pltpu.VMEM((32,), jnp.int32),
                                  pltpu.VMEM((32,), jnp.float32),
                                  pltpu.VMEM((32,), jnp.float32),
                                  pltpu.VMEM((32, DIM), jnp.float32),
                                  pltpu.VMEM((32, DIM), jnp.float32),
                                  pltpu.VMEM((32, DIM), jnp.float32),
                                  pltpu.SemaphoreType.DMA,
                                  pltpu.SemaphoreType.DMA])
def _sc_combine(ys_hbm, pos_hbm, w0_hbm, w1_hbm, y_hbm,
                p0v, p1v, w0v, w1v, a_v, b_v, o_v, sem0, sem1):
    _combine_body(ys_hbm, pos_hbm, w0_hbm, w1_hbm, y_hbm,
                  p0v, p1v, w0v, w1v, a_v, b_v, o_v, sem0, sem1)


def kernel(x, Wg, W1, W2):
    e1, e2, w0, w1 = _router(x, Wg)
    eidx = jnp.concatenate([e1.reshape(-1), e2.reshape(-1)])
    cnts = _sc_counts(eidx)
    xs, pos, sb = _sc_dispatch(eidx, cnts, x)
    ys = _ffn(sb, xs, W1, W2)
    return _sc_combine(ys, pos, w0.reshape(-1), w1.reshape(-1))


# speedup vs baseline: 1.2421x; 1.0065x over previous
import functools

import jax
import jax.numpy as jnp
from jax import lax
from jax.experimental import pallas as pl
from jax.experimental.pallas import tpu as pltpu
from jax.experimental.pallas import tpu_sc as plsc

N_TOK = 2048
DIM = 1024
N_EXP = 8
D_FF = 1024
N_PAIR = 2 * N_TOK
BM = 128
NBLK = 40
PAD = NBLK * BM
NW = 32
CHUNK = N_PAIR // NW
TOKW = N_TOK // NW

_MESH = plsc.VectorSubcoreMesh(core_axis_name="c", subcore_axis_name="s",
                               num_cores=2, num_subcores=16)


def _wid():
    return lax.axis_index("s") * 2 + lax.axis_index("c")


def _bi(s):
    return lax.broadcast(jnp.asarray(s, jnp.int32), (16,))


def _router_body(x_ref, wg_ref, e1_ref, e2_ref, w0_ref, w1_ref):
    x = x_ref[...]
    logits = lax.dot_general(x, wg_ref[...], (((1,), (1,)), ((), ())),
                             preferred_element_type=jnp.float32)
    m = jnp.max(logits, axis=1, keepdims=True)
    ex = jnp.exp(logits - m)
    p = ex / jnp.sum(ex, axis=1, keepdims=True)
    iota = lax.broadcasted_iota(jnp.int32, p.shape, 1)
    m1 = jnp.max(p, axis=1, keepdims=True)
    i1 = jnp.min(jnp.where(p == m1, iota, N_EXP), axis=1, keepdims=True)
    p2 = jnp.where(iota == i1, -jnp.inf, p)
    m2 = jnp.max(p2, axis=1, keepdims=True)
    i2 = jnp.min(jnp.where(p2 == m2, iota, N_EXP), axis=1, keepdims=True)
    s = m1 + m2
    e1_ref[...] = i1
    e2_ref[...] = i2
    w0_ref[...] = m1 / s
    w1_ref[...] = m2 / s


def _router(x, Wg):
    return pl.pallas_call(
        _router_body,
        in_specs=[pl.BlockSpec((N_TOK, DIM), lambda: (0, 0)),
                  pl.BlockSpec((N_EXP, DIM), lambda: (0, 0))],
        out_specs=[pl.BlockSpec((N_TOK, 1), lambda: (0, 0))] * 4,
        out_shape=[jax.ShapeDtypeStruct((N_TOK, 1), jnp.int32),
                   jax.ShapeDtypeStruct((N_TOK, 1), jnp.int32),
                   jax.ShapeDtypeStruct((N_TOK, 1), jnp.float32),
                   jax.ShapeDtypeStruct((N_TOK, 1), jnp.float32)],
    )(x, Wg)


def _counts_body(eidx_hbm, cnt_hbm, ev, crow):
    w = _wid()
    pltpu.sync_copy(eidx_hbm.at[pl.ds(w * CHUNK, CHUNK)], ev)
    lane = lax.iota(jnp.int32, 16)
    cnt = jnp.zeros((16,), jnp.int32)
    for i in range(CHUNK // 16):
        v = ev[pl.ds(i * 16, 16)]
        for e in range(N_EXP):
            c = jnp.sum(jnp.where(v == _bi(e), _bi(1), _bi(0)))
            cnt = jnp.where(lane == _bi(e), cnt + _bi(c), cnt)
    crow[...] = cnt
    pltpu.sync_copy(crow, cnt_hbm.at[w])


_SC_PARAMS = pltpu.CompilerParams(needs_layout_passes=False)


@functools.partial(pl.kernel,
                   out_type=jax.ShapeDtypeStruct((NW, 16), jnp.int32),
                   mesh=_MESH,
                   compiler_params=_SC_PARAMS,
                   scratch_types=[pltpu.VMEM((CHUNK,), jnp.int32),
                                  pltpu.VMEM((16,), jnp.int32)])
def _sc_counts(eidx_hbm, cnt_hbm, ev, crow):
    _counts_body(eidx_hbm, cnt_hbm, ev, crow)


def _dispatch_body(eidx_hbm, cnt_hbm, x_hbm, xs_hbm, pos_hbm, sb_hbm,
                   cnts_v, ev, base_v, posm, posl, sbv, rows_v, sem):
    w = _wid()
    pltpu.sync_copy(cnt_hbm, cnts_v)
    pltpu.sync_copy(eidx_hbm.at[pl.ds(w * CHUNK, CHUNK)], ev)
    lane = lax.iota(jnp.int32, 16)
    tot = jnp.zeros((16,), jnp.int32)
    pre = jnp.zeros((16,), jnp.int32)
    for t in range(NW):
        row = cnts_v[t]
        tot = tot + row
        pre = pre + row * _bi(jnp.where(t < w, 1, 0))
    padded = ((tot + _bi(BM - 1)) // _bi(BM)) * _bi(BM)
    starts = plsc.cumsum(padded) - padded
    base = starts + pre
    base_v[...] = base

    @pl.when(w == 0)
    def _():
        sbv[...] = starts // _bi(BM)
        pltpu.sync_copy(sbv, sb_hbm)

    for i in range(CHUNK // 16):
        v = ev[pl.ds(i * 16, 16)]
        bpl = plsc.load_gather(base_v, [v])
        rank = jnp.zeros((16,), jnp.int32)
        binc = jnp.zeros((16,), jnp.int32)
        for e in range(N_EXP):
            msk = v == _bi(e)
            mi = jnp.where(msk, _bi(1), _bi(0))
            cm = plsc.cumsum(mi)
            rank = jnp.where(msk, cm - _bi(1), rank)
            binc = jnp.where(lane == _bi(e), binc + _bi(jnp.sum(mi)), binc)
        pos_v = bpl + rank
        posm[i // 2, pl.ds((i % 2) * 16, 16)] = pos_v
        posl[pl.ds(i * 16, 16)] = pos_v
        base_v[...] = base_v[...] + binc

    pltpu.sync_copy(posl, pos_hbm.at[pl.ds(w * CHUNK, CHUNK)])
    tokbase = (w % 16) * CHUNK
    cps = []
    for c in range(4):
        if c >= 2:
            cps[c - 2].wait()
        pltpu.sync_copy(x_hbm.at[pl.ds(tokbase + c * 32, 32)],
                        rows_v.at[c % 2])
        cps.append(pltpu.async_copy(rows_v.at[c % 2],
                                    xs_hbm.at[posm.at[c]], sem))
    cps[2].wait()
    cps[3].wait()


@functools.partial(pl.kernel,
                   out_type=[jax.ShapeDtypeStruct((PAD, DIM), jnp.float32),
                             jax.ShapeDtypeStruct((N_PAIR,), jnp.int32),
                             jax.ShapeDtypeStruct((16,), jnp.int32)],
                   mesh=_MESH,
                   compiler_params=_SC_PARAMS,
                   scratch_types=[pltpu.VMEM((NW, 16), jnp.int32),
                                  pltpu.VMEM((CHUNK,), jnp.int32),
                                  pltpu.VMEM((16,), jnp.int32),
                                  pltpu.VMEM((4, 32), jnp.int32),
                                  pltpu.VMEM((CHUNK,), jnp.int32),
                                  pltpu.VMEM((16,), jnp.int32),
                                  pltpu.VMEM((2, 32, DIM), jnp.float32),
                                  pltpu.SemaphoreType.DMA])
def _sc_dispatch(eidx_hbm, cnt_hbm, x_hbm, xs_hbm, pos_hbm, sb_hbm,
                 cnts_v, ev, base_v, posm, posl, sbv, rows_v, sem):
    _dispatch_body(eidx_hbm, cnt_hbm, x_hbm, xs_hbm, pos_hbm, sb_hbm,
                   cnts_v, ev, base_v, posm, posl, sbv, rows_v, sem)


def _silu(h):
    return h * (1.0 / (1.0 + jnp.exp(-h)))


def _ffn_body(sb_ref, xs_ref, w1_ref, w2_ref, ys_ref):
    e = pl.program_id(0)
    lo = sb_ref[e]
    hi = sb_ref[e + 1]

    def _step(b, carry):
        xb = xs_ref[pl.ds(b * BM, BM), :]
        h = _silu(jnp.dot(xb, w1_ref[0], preferred_element_type=jnp.float32))
        ys_ref[pl.ds(b * BM, BM), :] = jnp.dot(
            h, w2_ref[0], preferred_element_type=jnp.float32)
        return carry

    lax.fori_loop(lo, hi, _step, 0)


def _ffn(sb, xs, W1, W2):
    grid_spec = pltpu.PrefetchScalarGridSpec(
        num_scalar_prefetch=1,
        grid=(N_EXP,),
        in_specs=[
            pl.BlockSpec((PAD, DIM), lambda e, sb: (0, 0)),
            pl.BlockSpec((1, DIM, D_FF), lambda e, sb: (e, 0, 0)),
            pl.BlockSpec((1, D_FF, DIM), lambda e, sb: (e, 0, 0)),
        ],
        out_specs=pl.BlockSpec((PAD, DIM), lambda e, sb: (0, 0)),
    )
    return pl.pallas_call(
        _ffn_body,
        grid_spec=grid_spec,
        out_shape=jax.ShapeDtypeStruct((PAD, DIM), jnp.float32),
    )(sb, xs, W1, W2)


def _combine_body(ys_hbm, pos_hbm, w0_hbm, w1_hbm, y_hbm,
                  p0v, p1v, w0v, w1v, a_v, b_v, o_v, sem0, sem1):
    w = _wid()
    for c in range(TOKW // 32):
        tb = w * TOKW + c * 32
        pltpu.sync_copy(pos_hbm.at[pl.ds(tb, 32)], p0v)
        pltpu.sync_copy(pos_hbm.at[pl.ds(N_TOK + tb, 32)], p1v)
        pltpu.sync_copy(w0_hbm.at[pl.ds(tb, 32)], w0v)
        pltpu.sync_copy(w1_hbm.at[pl.ds(tb, 32)], w1v)
        cpa = pltpu.async_copy(ys_hbm.at[p0v], a_v, sem0)
        cpb = pltpu.async_copy(ys_hbm.at[p1v], b_v, sem1)
        cpa.wait()
        cpb.wait()

        def _row(t, carry):
            ts = _bi(t)
            w0s = plsc.load_gather(w0v, [ts])
            w1s = plsc.load_gather(w1v, [ts])
            for j in range(DIM // 16):
                sl = pl.ds(j * 16, 16)
                o_v[t, sl] = a_v[t, sl] * w0s + b_v[t, sl] * w1s
            return carry

        lax.fori_loop(0, 32, _row, 0)
        pltpu.sync_copy(o_v, y_hbm.at[pl.ds(tb, 32)])


@functools.partial(pl.kernel,
                   out_type=jax.ShapeDtypeStruct((N_TOK, DIM), jnp.float32),
                   mesh=_MESH,
                   compiler_params=_SC_PARAMS,
                   scratch_types=[pltpu.VMEM((32,), jnp.int32),
                                  pltpu.VMEM((32,), jnp.int32),
                                  pltpu.VMEM((32,), jnp.float32),
                                  pltpu.VMEM((32,), jnp.float32),
                                  pltpu.VMEM((32, DIM), jnp.float32),
                                  pltpu.VMEM((32, DIM), jnp.float32),
                                  pltpu.VMEM((32, DIM), jnp.float32),
                                  pltpu.SemaphoreType.DMA,
                                  pltpu.SemaphoreType.DMA])
def _sc_combine(ys_hbm, pos_hbm, w0_hbm, w1_hbm, y_hbm,
                p0v, p1v, w0v, w1v, a_v, b_v, o_v, sem0, sem1):
    _combine_body(ys_hbm, pos_hbm, w0_hbm, w1_hbm, y_hbm,
                  p0v, p1v, w0v, w1v, a_v, b_v, o_v, sem0, sem1)


def kernel(x, Wg, W1, W2):
    e1, e2, w0, w1 = _router(x, Wg)
    eidx = jnp.concatenate([e1.reshape(-1), e2.reshape(-1)])
    cnts = _sc_counts(eidx)
    xs, pos, sb = _sc_dispatch(eidx, cnts, x)
    ys = _ffn(sb, xs, W1, W2)
    return _sc_combine(ys, pos, w0.reshape(-1), w1.reshape(-1))
